# baseline (speedup 1.0000x reference)
import functools

import jax
import jax.numpy as jnp
from jax import lax
from jax.experimental import pallas as pl
from jax.experimental.pallas import tpu as pltpu
from jax.experimental.pallas import tpu_sc as plsc

NUM_VALUES = 1000000
EMBED_DIM = 16
BATCH = 16384

_INFO = plsc.get_sparse_core_info()
_NC, _NS, _L = _INFO.num_cores, _INFO.num_subcores, _INFO.num_lanes
_NW = _NC * _NS
_B_PER_W = BATCH // _NW
_CHUNK = 128
_N_CHUNKS = _B_PER_W // _CHUNK


def _make_kernel():
    mesh = plsc.VectorSubcoreMesh(core_axis_name="c", subcore_axis_name="s")

    @functools.partial(
        pl.kernel,
        mesh=mesh,
        out_type=jax.ShapeDtypeStruct((BATCH, EMBED_DIM), jnp.float32),
        scratch_types=[
            pltpu.VMEM((_B_PER_W,), jnp.int32),
            pltpu.VMEM((_B_PER_W, EMBED_DIM), jnp.float32),
            pltpu.SemaphoreType.DMA,
        ],
        compiler_params=pltpu.CompilerParams(use_tc_tiling_on_sc=False),
    )
    def gather_kernel(idx_hbm, table_hbm, out_hbm, idx_v, rows_v, sem):
        wid = lax.axis_index("s") * _NC + lax.axis_index("c")
        base = wid * _B_PER_W

        pltpu.sync_copy(idx_hbm.at[pl.ds(base, _B_PER_W)], idx_v)

        for k in range(_B_PER_W // _L):
            v = idx_v[pl.ds(k * _L, _L)]
            idx_v[pl.ds(k * _L, _L)] = jnp.where(v >= NUM_VALUES, 0, v)

        copies = []
        for j in range(_N_CHUNKS):
            copies.append(
                pltpu.async_copy(
                    table_hbm.at[idx_v.at[pl.ds(j * _CHUNK, _CHUNK)]],
                    rows_v.at[pl.ds(j * _CHUNK, _CHUNK)],
                    sem,
                )
            )
        for c in copies:
            c.wait()

        pltpu.sync_copy(rows_v, out_hbm.at[pl.ds(base, _B_PER_W)])

    return gather_kernel


_GATHER = _make_kernel()


def kernel(categorical_column, table):
    idx = categorical_column.astype(jnp.int32)
    return _GATHER(idx, table)
